# Initial kernel scaffold; baseline (speedup 1.0000x reference)
import jax
import jax.numpy as jnp
from jax.experimental import pallas as pl


def kernel(x, W_router):
    raise NotImplementedError("write your pallas kernel here")


# speedup vs baseline: 3.7280x; 3.7280x over previous
import functools

import jax
import jax.numpy as jnp
import numpy as np
from jax.experimental import pallas as pl
from jax.experimental.pallas import tpu as pltpu

_MININT = np.int32(-(2 ** 31))


def _sortable(bits):
    return jnp.where(bits >= 0, bits, bits ^ np.int32(0x7FFFFFFF))


def _stats_kernel(x_ref, w_ref, stats_ref, scores_ref, *, ns, k):
    i = pl.program_id(1)
    xb = x_ref[0]
    s_row = jax.lax.dot_general(
        w_ref[...], xb, (((1,), (1,)), ((), ())),
        preferred_element_type=jnp.float32)
    scores_ref[pl.ds(i, 1), :] = s_row

    @pl.when(i == ns - 1)
    def _():
        sc = scores_ref[...]
        key = _sortable(jax.lax.bitcast_convert_type(sc, jnp.int32))

        def body(j, ts):
            b = 31 - j
            m = jnp.left_shift(np.int32(1), b)
            cand = jnp.where(b == 31, np.int32(0), ts | m)
            cnt = jnp.sum((key >= cand).astype(jnp.int32))
            return jnp.where(cnt >= k, cand, ts)

        ts = jax.lax.fori_loop(0, 32, body, _MININT)
        mx = jnp.max(sc)
        z = jnp.sum(jnp.where(key >= ts, jnp.exp(sc - mx), 0.0))
        lane = jax.lax.broadcasted_iota(jnp.int32, (1, 1, 128), 2)
        ts_f = jax.lax.bitcast_convert_type(
            jnp.full((1, 1, 128), ts, jnp.int32), jnp.float32)
        stats_ref[...] = jnp.where(
            lane == 0, ts_f, jnp.where(lane == 1, mx, z))


def _apply_kernel(x_ref, w_ref, stats_ref, o_ref):
    xb = x_ref[0]
    s = jax.lax.dot_general(
        xb, w_ref[...], (((1,), (1,)), ((), ())),
        preferred_element_type=jnp.float32)
    stats = stats_ref[0]
    ts = jax.lax.bitcast_convert_type(stats[0:1, 0:1], jnp.int32)
    mx = stats[0:1, 1:2]
    z = stats[0:1, 2:3]
    key = _sortable(jax.lax.bitcast_convert_type(s, jnp.int32))
    mask = key >= ts
    wgt = jnp.exp(s - mx) / z
    vals = wgt * xb + (1.0 - wgt) * xb
    o_ref[0] = jnp.where(mask, vals, xb)


def kernel(x, W_router):
    B, S, D = x.shape
    k = max(1, int(S * 0.75))
    SB = 512
    NS = S // SB

    stats = pl.pallas_call(
        functools.partial(_stats_kernel, ns=NS, k=k),
        grid=(B, NS),
        in_specs=[
            pl.BlockSpec((1, SB, D), lambda b, i: (b, i, 0)),
            pl.BlockSpec((1, D), lambda b, i: (0, 0)),
        ],
        out_specs=pl.BlockSpec((1, 1, 128), lambda b, i: (b, 0, 0)),
        out_shape=jax.ShapeDtypeStruct((B, 1, 128), jnp.float32),
        scratch_shapes=[pltpu.VMEM((NS, SB), jnp.float32)],
        compiler_params=pltpu.CompilerParams(
            dimension_semantics=("arbitrary", "arbitrary")),
    )(x, W_router)

    out = pl.pallas_call(
        _apply_kernel,
        grid=(B, NS),
        in_specs=[
            pl.BlockSpec((1, SB, D), lambda b, i: (b, i, 0)),
            pl.BlockSpec((1, D), lambda b, i: (0, 0)),
            pl.BlockSpec((1, 1, 128), lambda b, i: (b, 0, 0)),
        ],
        out_specs=pl.BlockSpec((1, SB, D), lambda b, i: (b, i, 0)),
        out_shape=jax.ShapeDtypeStruct((B, S, D), jnp.float32),
        compiler_params=pltpu.CompilerParams(
            dimension_semantics=("parallel", "arbitrary")),
    )(x, W_router, stats)
    return out


# speedup vs baseline: 4.8091x; 1.2900x over previous
import functools

import jax
import jax.numpy as jnp
import numpy as np
from jax.experimental import pallas as pl
from jax.experimental.pallas import tpu as pltpu

_MININT = np.int32(-(2 ** 31))


def _sortable(bits):
    return jnp.where(bits >= 0, bits, bits ^ np.int32(0x7FFFFFFF))


def _stats_kernel(x_ref, w_ref, stats_ref, scores_ref, *, ns, k):
    i = pl.program_id(1)
    xb = x_ref[0]
    s_row = jax.lax.dot_general(
        w_ref[...], xb, (((1,), (1,)), ((), ())),
        preferred_element_type=jnp.float32)
    scores_ref[pl.ds(i, 1), :] = s_row

    @pl.when(i == ns - 1)
    def _():
        sc = scores_ref[...]
        key = _sortable(jax.lax.bitcast_convert_type(sc, jnp.int32))

        def body(j, ts):
            b = 31 - j
            m = jnp.left_shift(np.int32(1), b)
            cand = jnp.where(b == 31, np.int32(0), ts | m)
            cnt = jnp.sum((key >= cand).astype(jnp.int32))
            return jnp.where(cnt >= k, cand, ts)

        ts = jax.lax.fori_loop(0, 32, body, _MININT)
        mx = jnp.max(sc)
        z = jnp.sum(jnp.where(key >= ts, jnp.exp(sc - mx), 0.0))
        lane = jax.lax.broadcasted_iota(jnp.int32, (1, 1, 128), 2)
        ts_f = jax.lax.bitcast_convert_type(
            jnp.full((1, 1, 128), ts, jnp.int32), jnp.float32)
        stats_ref[...] = jnp.where(
            lane == 0, ts_f, jnp.where(lane == 1, mx, z))


def _apply_kernel(x_ref, w_ref, stats_ref, o_ref):
    xb = x_ref[0]
    s = jax.lax.dot_general(
        xb, w_ref[...], (((1,), (1,)), ((), ())),
        preferred_element_type=jnp.float32)
    stats = stats_ref[0]
    ts = jax.lax.bitcast_convert_type(stats[0:1, 0:1], jnp.int32)
    mx = stats[0:1, 1:2]
    z = stats[0:1, 2:3]
    key = _sortable(jax.lax.bitcast_convert_type(s, jnp.int32))
    mask = key >= ts
    wgt = jnp.exp(s - mx) / z
    vals = wgt * xb + (1.0 - wgt) * xb
    o_ref[0] = jnp.where(mask, vals, xb)


def kernel(x, W_router):
    B, S, D = x.shape
    k = max(1, int(S * 0.75))
    SB = 2048
    NS = S // SB

    stats = pl.pallas_call(
        functools.partial(_stats_kernel, ns=NS, k=k),
        grid=(B, NS),
        in_specs=[
            pl.BlockSpec((1, SB, D), lambda b, i: (b, i, 0)),
            pl.BlockSpec((1, D), lambda b, i: (0, 0)),
        ],
        out_specs=pl.BlockSpec((1, 1, 128), lambda b, i: (b, 0, 0)),
        out_shape=jax.ShapeDtypeStruct((B, 1, 128), jnp.float32),
        scratch_shapes=[pltpu.VMEM((NS, SB), jnp.float32)],
        compiler_params=pltpu.CompilerParams(
            dimension_semantics=("arbitrary", "arbitrary")),
    )(x, W_router)

    out = pl.pallas_call(
        _apply_kernel,
        grid=(B, NS),
        in_specs=[
            pl.BlockSpec((1, SB, D), lambda b, i: (b, i, 0)),
            pl.BlockSpec((1, D), lambda b, i: (0, 0)),
            pl.BlockSpec((1, 1, 128), lambda b, i: (b, 0, 0)),
        ],
        out_specs=pl.BlockSpec((1, SB, D), lambda b, i: (b, i, 0)),
        out_shape=jax.ShapeDtypeStruct((B, S, D), jnp.float32),
        compiler_params=pltpu.CompilerParams(
            dimension_semantics=("parallel", "parallel")),
    )(x, W_router, stats)
    return out
